# baseline (speedup 1.0000x reference)
import functools

import jax
import jax.numpy as jnp
from jax import lax
from jax.experimental import pallas as pl
from jax.experimental.pallas import tpu as pltpu
from jax.experimental.pallas import tpu_sc as plsc

DECAY_K = 0.95
CHUNK = 2048
SUB = 128


def _tc_body(mem_ref, cnt_ref, content_ref, gate_ref, out_ref):
    val = content_ref[0] * gate_ref[0]
    cnt = cnt_ref[0]
    na = CHUNK // SUB
    for u in range(na):
        sl = pl.ds(u * SUB, SUB)
        col = cnt[:, u:u + 1]
        out_ref[0, sl, :] = mem_ref[0, sl, :] * DECAY_K + col * val


def _tc_apply(memory_state, cnt3, content3, gate3):
    B, J, D = memory_state.shape
    nch = J // CHUNK
    grid = (B, nch)
    return pl.pallas_call(
        _tc_body,
        grid=grid,
        in_specs=[
            pl.BlockSpec((1, CHUNK, D), lambda b, c: (b, c, 0)),
            pl.BlockSpec((1, SUB, CHUNK // SUB), lambda b, c: (b * nch + c, 0, 0)),
            pl.BlockSpec((1, 1, D), lambda b, c: (b, 0, 0)),
            pl.BlockSpec((1, 1, 1), lambda b, c: (b, 0, 0)),
        ],
        out_specs=pl.BlockSpec((1, CHUNK, D), lambda b, c: (b, c, 0)),
        out_shape=jax.ShapeDtypeStruct((B, J, D), jnp.float32),
    )(memory_state, cnt3, content3, gate3)


def _make_sc_hist(B, W, J):
    info = plsc.get_sparse_core_info()
    nc, ns = info.num_cores, info.num_subcores
    nw = nc * ns
    bpw = B // nw
    ib = (SUB - 1).bit_length()
    ub = (CHUNK // SUB - 1).bit_length()
    mesh = plsc.VectorSubcoreMesh(core_axis_name="c", subcore_axis_name="s")

    @functools.partial(
        pl.kernel,
        out_type=jax.ShapeDtypeStruct((B, J), jnp.float32),
        mesh=mesh,
        compiler_params=pltpu.CompilerParams(needs_layout_passes=False),
        scratch_types=[
            pltpu.VMEM((W,), jnp.int32),
            pltpu.VMEM((J,), jnp.float32),
        ],
    )
    def sc_hist(idx_hbm, zeros_hbm, out_hbm, idx_v, counts_v):
        wid = lax.axis_index("s") * nc + lax.axis_index("c")
        ones16 = jnp.ones((16,), jnp.float32)
        for k in range(bpw):
            b = wid * bpw + k
            pltpu.sync_copy(idx_hbm.at[b], idx_v)
            pltpu.sync_copy(zeros_hbm, counts_v)
            for ci in range(W // 16):
                v = idx_v[pl.ds(ci * 16, 16)]
                vt = (((v >> (ib + ub)) << (ib + ub))
                      | ((v & (SUB - 1)) << ub)
                      | ((v >> ib) & (CHUNK // SUB - 1)))
                plsc.addupdate_scatter(counts_v, [vt], ones16)
            pltpu.sync_copy(counts_v, out_hbm.at[b])

    return sc_hist


def kernel(memory_state, write_indices, write_content, gate):
    B, J, D = memory_state.shape
    W = write_indices.shape[1]
    idx = write_indices.astype(jnp.int32)
    zeros = jnp.zeros((J,), jnp.float32)
    counts_t = _make_sc_hist(B, W, J)(idx, zeros)
    cnt3 = counts_t.reshape(B * (J // CHUNK), SUB, CHUNK // SUB)
    content3 = write_content.reshape(B, 1, D)
    gate3 = gate.reshape(B, 1, 1)
    return _tc_apply(memory_state, cnt3, content3, gate3)


# speedup vs baseline: 5.6009x; 5.6009x over previous
import functools

import jax
import jax.numpy as jnp
from jax import lax
from jax.experimental import pallas as pl
from jax.experimental.pallas import tpu as pltpu
from jax.experimental.pallas import tpu_sc as plsc

DECAY_K = 0.95
CJ = 8192


def _tc_body(mem_ref, cnt_ref, content_ref, gate_ref, out_ref):
    d = content_ref.shape[2]
    val = (content_ref[0] * gate_ref[0]).reshape(d, 1)
    out_ref[0] = mem_ref[0] * DECAY_K + cnt_ref[0] * val


def _tc_apply(mem_t, cnt3, content3, gate3):
    B, D, J = mem_t.shape
    grid = (B, J // CJ)
    return pl.pallas_call(
        _tc_body,
        grid=grid,
        in_specs=[
            pl.BlockSpec((1, D, CJ), lambda b, c: (b, 0, c)),
            pl.BlockSpec((1, 1, CJ), lambda b, c: (b, 0, c)),
            pl.BlockSpec((1, 1, D), lambda b, c: (b, 0, 0)),
            pl.BlockSpec((1, 1, 1), lambda b, c: (b, 0, 0)),
        ],
        out_specs=pl.BlockSpec((1, D, CJ), lambda b, c: (b, 0, c)),
        out_shape=jax.ShapeDtypeStruct((B, D, J), jnp.float32),
    )(mem_t, cnt3, content3, gate3)


def _make_sc_hist(B, W, J):
    info = plsc.get_sparse_core_info()
    nc, ns = info.num_cores, info.num_subcores
    nw = nc * ns
    bpw = B // nw
    mesh = plsc.VectorSubcoreMesh(core_axis_name="c", subcore_axis_name="s")

    @functools.partial(
        pl.kernel,
        out_type=jax.ShapeDtypeStruct((B, J), jnp.float32),
        mesh=mesh,
        compiler_params=pltpu.CompilerParams(needs_layout_passes=False),
        scratch_types=[
            pltpu.VMEM((W,), jnp.int32),
            pltpu.VMEM((J,), jnp.float32),
        ],
    )
    def sc_hist(idx_hbm, zeros_hbm, out_hbm, idx_v, counts_v):
        wid = lax.axis_index("s") * nc + lax.axis_index("c")
        ones16 = jnp.ones((16,), jnp.float32)
        for k in range(bpw):
            b = wid * bpw + k
            pltpu.sync_copy(idx_hbm.at[b], idx_v)
            pltpu.sync_copy(zeros_hbm, counts_v)
            for ci in range(W // 16):
                v = idx_v[pl.ds(ci * 16, 16)]
                plsc.addupdate_scatter(counts_v, [v], ones16)
            pltpu.sync_copy(counts_v, out_hbm.at[b])

    return sc_hist


def kernel(memory_state, write_indices, write_content, gate):
    B, J, D = memory_state.shape
    W = write_indices.shape[1]
    idx = write_indices.astype(jnp.int32)
    zeros = jnp.zeros((J,), jnp.float32)
    counts = _make_sc_hist(B, W, J)(idx, zeros)
    cnt3 = counts.reshape(B, 1, J)
    content3 = write_content.reshape(B, 1, D)
    gate3 = gate.reshape(B, 1, 1)
    mem_t = jnp.transpose(memory_state, (0, 2, 1))
    out_t = _tc_apply(mem_t, cnt3, content3, gate3)
    return jnp.transpose(out_t, (0, 2, 1))


# speedup vs baseline: 6.3611x; 1.1357x over previous
import functools

import jax
import jax.numpy as jnp
from jax import lax
from jax.experimental import pallas as pl
from jax.experimental.pallas import tpu as pltpu
from jax.experimental.pallas import tpu_sc as plsc

DECAY_K = 0.95
CJ = 16384


def _tc_body(mem_ref, cnt_ref, content_ref, gate_ref, out_ref):
    d = content_ref.shape[2]
    val = (content_ref[0] * gate_ref[0]).reshape(d, 1)
    out_ref[0] = mem_ref[0] * DECAY_K + cnt_ref[0] * val


def _tc_apply(mem_t, cnt3, content3, gate3):
    B, D, J = mem_t.shape
    grid = (B, J // CJ)
    return pl.pallas_call(
        _tc_body,
        grid=grid,
        in_specs=[
            pl.BlockSpec((1, D, CJ), lambda b, c: (b, 0, c)),
            pl.BlockSpec((1, 1, CJ), lambda b, c: (b, 0, c)),
            pl.BlockSpec((1, 1, D), lambda b, c: (b, 0, 0)),
            pl.BlockSpec((1, 1, 1), lambda b, c: (b, 0, 0)),
        ],
        out_specs=pl.BlockSpec((1, D, CJ), lambda b, c: (b, 0, c)),
        out_shape=jax.ShapeDtypeStruct((B, D, J), jnp.float32),
    )(mem_t, cnt3, content3, gate3)


def _make_sc_hist(B, W, J):
    info = plsc.get_sparse_core_info()
    nc, ns = info.num_cores, info.num_subcores
    nw = nc * ns
    bpw = B // nw
    mesh = plsc.VectorSubcoreMesh(core_axis_name="c", subcore_axis_name="s")

    @functools.partial(
        pl.kernel,
        out_type=jax.ShapeDtypeStruct((B, J), jnp.float32),
        mesh=mesh,
        compiler_params=pltpu.CompilerParams(needs_layout_passes=False),
        scratch_types=[
            pltpu.VMEM((W,), jnp.int32),
            pltpu.VMEM((J,), jnp.float32),
        ],
    )
    def sc_hist(idx_hbm, zeros_hbm, out_hbm, idx_v, counts_v):
        wid = lax.axis_index("s") * nc + lax.axis_index("c")
        ones16 = jnp.ones((16,), jnp.float32)
        for k in range(bpw):
            b = wid * bpw + k
            pltpu.sync_copy(idx_hbm.at[b], idx_v)
            pltpu.sync_copy(zeros_hbm, counts_v)
            for ci in range(W // 16):
                v = idx_v[pl.ds(ci * 16, 16)]
                plsc.addupdate_scatter(counts_v, [v], ones16)
            pltpu.sync_copy(counts_v, out_hbm.at[b])

    return sc_hist


def kernel(memory_state, write_indices, write_content, gate):
    B, J, D = memory_state.shape
    W = write_indices.shape[1]
    idx = write_indices.astype(jnp.int32)
    zeros = jnp.zeros((J,), jnp.float32)
    counts = _make_sc_hist(B, W, J)(idx, zeros)
    cnt3 = counts.reshape(B, 1, J)
    content3 = write_content.reshape(B, 1, D)
    gate3 = gate.reshape(B, 1, 1)
    mem_t = jnp.transpose(memory_state, (0, 2, 1))
    out_t = _tc_apply(mem_t, cnt3, content3, gate3)
    return jnp.transpose(out_t, (0, 2, 1))


# speedup vs baseline: 6.4953x; 1.0211x over previous
import functools

import jax
import jax.numpy as jnp
from jax import lax
from jax.experimental import pallas as pl
from jax.experimental.pallas import tpu as pltpu
from jax.experimental.pallas import tpu_sc as plsc

DECAY_K = 0.95
CJ = 16384


BB = 2


def _tc_body(mem_ref, cnt_ref, content_ref, gate_ref, out_ref):
    d = content_ref.shape[2]
    for i in range(BB):
        val = (content_ref[i] * gate_ref[i]).reshape(d, 1)
        out_ref[i] = mem_ref[i] * DECAY_K + cnt_ref[i] * val


def _tc_apply(mem_t, cnt3, content3, gate3):
    B, D, J = mem_t.shape
    grid = (B // BB, J // CJ)
    return pl.pallas_call(
        _tc_body,
        grid=grid,
        in_specs=[
            pl.BlockSpec((BB, D, CJ), lambda b, c: (b, 0, c)),
            pl.BlockSpec((BB, 1, CJ), lambda b, c: (b, 0, c)),
            pl.BlockSpec((BB, 1, D), lambda b, c: (b, 0, 0)),
            pl.BlockSpec((BB, 1, 1), lambda b, c: (b, 0, 0)),
        ],
        out_specs=pl.BlockSpec((BB, D, CJ), lambda b, c: (b, 0, c)),
        out_shape=jax.ShapeDtypeStruct((B, D, J), jnp.float32),
    )(mem_t, cnt3, content3, gate3)


def _make_sc_hist(B, W, J):
    info = plsc.get_sparse_core_info()
    nc, ns = info.num_cores, info.num_subcores
    nw = nc * ns
    bpw = B // nw
    mesh = plsc.VectorSubcoreMesh(core_axis_name="c", subcore_axis_name="s")

    @functools.partial(
        pl.kernel,
        out_type=jax.ShapeDtypeStruct((B, J), jnp.float32),
        mesh=mesh,
        compiler_params=pltpu.CompilerParams(needs_layout_passes=False),
        scratch_types=[
            pltpu.VMEM((W,), jnp.int32),
            pltpu.VMEM((J,), jnp.float32),
        ],
    )
    def sc_hist(idx_hbm, zeros_hbm, out_hbm, idx_v, counts_v):
        wid = lax.axis_index("s") * nc + lax.axis_index("c")
        ones16 = jnp.ones((16,), jnp.float32)
        for k in range(bpw):
            b = wid * bpw + k
            pltpu.sync_copy(idx_hbm.at[b], idx_v)
            pltpu.sync_copy(zeros_hbm, counts_v)
            for ci in range(W // 16):
                v = idx_v[pl.ds(ci * 16, 16)]
                plsc.addupdate_scatter(counts_v, [v], ones16)
            pltpu.sync_copy(counts_v, out_hbm.at[b])

    return sc_hist


def kernel(memory_state, write_indices, write_content, gate):
    B, J, D = memory_state.shape
    W = write_indices.shape[1]
    idx = write_indices.astype(jnp.int32)
    zeros = jnp.zeros((J,), jnp.float32)
    counts = _make_sc_hist(B, W, J)(idx, zeros)
    cnt3 = counts.reshape(B, 1, J)
    content3 = write_content.reshape(B, 1, D)
    gate3 = gate.reshape(B, 1, 1)
    mem_t = jnp.transpose(memory_state, (0, 2, 1))
    out_t = _tc_apply(mem_t, cnt3, content3, gate3)
    return jnp.transpose(out_t, (0, 2, 1))


# speedup vs baseline: 6.6006x; 1.0162x over previous
import functools

import jax
import jax.numpy as jnp
from jax import lax
from jax.experimental import pallas as pl
from jax.experimental.pallas import tpu as pltpu
from jax.experimental.pallas import tpu_sc as plsc

DECAY_K = 0.95
CJ = 16384


BB = 2


def _tc_body(mem_ref, cnt_ref, content_ref, gate_ref, out_ref):
    d = content_ref.shape[2]
    for i in range(BB):
        val = (content_ref[i] * gate_ref[i]).reshape(d, 1)
        out_ref[i] = mem_ref[i] * DECAY_K + cnt_ref[i] * val


def _tc_apply(mem_t, cnt3, content3, gate3):
    B, D, J = mem_t.shape
    grid = (B // BB, J // CJ)
    return pl.pallas_call(
        _tc_body,
        grid=grid,
        in_specs=[
            pl.BlockSpec((BB, D, CJ), lambda b, c: (b, 0, c)),
            pl.BlockSpec((BB, 1, CJ), lambda b, c: (b, 0, c)),
            pl.BlockSpec((BB, 1, D), lambda b, c: (b, 0, 0)),
            pl.BlockSpec((BB, 1, 1), lambda b, c: (b, 0, 0)),
        ],
        out_specs=pl.BlockSpec((BB, D, CJ), lambda b, c: (b, 0, c)),
        out_shape=jax.ShapeDtypeStruct((B, D, J), jnp.float32),
    )(mem_t, cnt3, content3, gate3)


def _make_sc_hist(B, W, J):
    info = plsc.get_sparse_core_info()
    nc, ns = info.num_cores, info.num_subcores
    nw = nc * ns
    bpw = B // nw
    mesh = plsc.VectorSubcoreMesh(core_axis_name="c", subcore_axis_name="s")

    @functools.partial(
        pl.kernel,
        out_type=jax.ShapeDtypeStruct((B, J), jnp.float32),
        mesh=mesh,
        compiler_params=pltpu.CompilerParams(needs_layout_passes=False),
        scratch_types=[
            pltpu.VMEM((bpw, W), jnp.int32),
            pltpu.VMEM((J,), jnp.float32),
            pltpu.SemaphoreType.DMA,
            pltpu.SemaphoreType.DMA,
        ],
    )
    def sc_hist(idx_hbm, zeros_hbm, out_hbm, idx_v, counts_v, sem_i, sem_o):
        wid = lax.axis_index("s") * nc + lax.axis_index("c")
        ones16 = jnp.ones((16,), jnp.float32)
        neg16 = jnp.full((16,), -1.0, jnp.float32)
        cp_i = pltpu.async_copy(
            idx_hbm.at[pl.ds(wid * bpw, bpw)], idx_v, sem_i)
        pltpu.sync_copy(zeros_hbm, counts_v)
        cp_i.wait()
        for k in range(bpw):
            b = wid * bpw + k
            for ci in range(W // 16):
                v = idx_v[k, pl.ds(ci * 16, 16)]
                plsc.addupdate_scatter(counts_v, [v], ones16)
            pltpu.async_copy(counts_v, out_hbm.at[b], sem_o).wait()
            if k + 1 < bpw:
                for ci in range(W // 16):
                    v = idx_v[k, pl.ds(ci * 16, 16)]
                    plsc.addupdate_scatter(counts_v, [v], neg16)

    return sc_hist


def kernel(memory_state, write_indices, write_content, gate):
    B, J, D = memory_state.shape
    W = write_indices.shape[1]
    idx = write_indices.astype(jnp.int32)
    zeros = jnp.zeros((J,), jnp.float32)
    counts = _make_sc_hist(B, W, J)(idx, zeros)
    cnt3 = counts.reshape(B, 1, J)
    content3 = write_content.reshape(B, 1, D)
    gate3 = gate.reshape(B, 1, 1)
    mem_t = jnp.transpose(memory_state, (0, 2, 1))
    out_t = _tc_apply(mem_t, cnt3, content3, gate3)
    return jnp.transpose(out_t, (0, 2, 1))


# speedup vs baseline: 6.7945x; 1.0294x over previous
import functools

import jax
import jax.numpy as jnp
from jax import lax
from jax.experimental import pallas as pl
from jax.experimental.pallas import tpu as pltpu
from jax.experimental.pallas import tpu_sc as plsc

DECAY_K = 0.95
CJ = 4096
BB = 8


def _tc_body(mem_ref, cnt_ref, content_ref, gate_ref, out_ref):
    d = content_ref.shape[2]
    for i in range(BB):
        val = (content_ref[i] * gate_ref[i]).reshape(d, 1)
        out_ref[i] = mem_ref[i] * DECAY_K + cnt_ref[i:i + 1, :] * val


def _tc_apply(mem_t, cnt, content3, gate3):
    B, D, J = mem_t.shape
    grid = (B // BB, J // CJ)
    return pl.pallas_call(
        _tc_body,
        grid=grid,
        in_specs=[
            pl.BlockSpec((BB, D, CJ), lambda b, c: (b, 0, c)),
            pl.BlockSpec((BB, CJ), lambda b, c: (b, c)),
            pl.BlockSpec((BB, 1, D), lambda b, c: (b, 0, 0)),
            pl.BlockSpec((BB, 1, 1), lambda b, c: (b, 0, 0)),
        ],
        out_specs=pl.BlockSpec((BB, D, CJ), lambda b, c: (b, 0, c)),
        out_shape=jax.ShapeDtypeStruct((B, D, J), jnp.float32),
    )(mem_t, cnt, content3, gate3)


def _make_sc_hist(B, W, J):
    info = plsc.get_sparse_core_info()
    nc, ns = info.num_cores, info.num_subcores
    nw = nc * ns
    bpw = B // nw
    mesh = plsc.VectorSubcoreMesh(core_axis_name="c", subcore_axis_name="s")

    @functools.partial(
        pl.kernel,
        out_type=jax.ShapeDtypeStruct((B, J), jnp.float32),
        mesh=mesh,
        compiler_params=pltpu.CompilerParams(needs_layout_passes=False),
        scratch_types=[
            pltpu.VMEM((bpw, W), jnp.int32),
            pltpu.VMEM((J,), jnp.float32),
            pltpu.SemaphoreType.DMA,
            pltpu.SemaphoreType.DMA,
        ],
    )
    def sc_hist(idx_hbm, zeros_hbm, out_hbm, idx_v, counts_v, sem_i, sem_o):
        wid = lax.axis_index("s") * nc + lax.axis_index("c")
        ones16 = jnp.ones((16,), jnp.float32)
        neg16 = jnp.full((16,), -1.0, jnp.float32)
        cp_i = pltpu.async_copy(
            idx_hbm.at[pl.ds(wid * bpw, bpw)], idx_v, sem_i)
        pltpu.sync_copy(zeros_hbm, counts_v)
        cp_i.wait()
        for k in range(bpw):
            b = wid * bpw + k
            for ci in range(W // 16):
                v = idx_v[k, pl.ds(ci * 16, 16)]
                plsc.addupdate_scatter(counts_v, [v], ones16)
            pltpu.async_copy(counts_v, out_hbm.at[b], sem_o).wait()
            if k + 1 < bpw:
                for ci in range(W // 16):
                    v = idx_v[k, pl.ds(ci * 16, 16)]
                    plsc.addupdate_scatter(counts_v, [v], neg16)

    return sc_hist


def kernel(memory_state, write_indices, write_content, gate):
    B, J, D = memory_state.shape
    W = write_indices.shape[1]
    idx = write_indices.astype(jnp.int32)
    zeros = jnp.zeros((J,), jnp.float32)
    counts = _make_sc_hist(B, W, J)(idx, zeros)
    content3 = write_content.reshape(B, 1, D)
    gate3 = gate.reshape(B, 1, 1)
    mem_t = jnp.transpose(memory_state, (0, 2, 1))
    out_t = _tc_apply(mem_t, counts, content3, gate3)
    return jnp.transpose(out_t, (0, 2, 1))


# speedup vs baseline: 6.8034x; 1.0013x over previous
import functools

import jax
import jax.numpy as jnp
from jax import lax
from jax.experimental import pallas as pl
from jax.experimental.pallas import tpu as pltpu
from jax.experimental.pallas import tpu_sc as plsc

DECAY_K = 0.95
CJ = 4096
BB = 8


def _tc_body(mem_ref, cnt_ref, content_ref, gate_ref, out_ref):
    d = content_ref.shape[2]
    for i in range(BB):
        val = (content_ref[i] * gate_ref[i]).reshape(d, 1)
        out_ref[i] = mem_ref[i] * DECAY_K + cnt_ref[i] * val


def _tc_apply(mem_t, cnt, content3, gate3):
    B, D, J = mem_t.shape
    grid = (B // BB, J // CJ)
    return pl.pallas_call(
        _tc_body,
        grid=grid,
        in_specs=[
            pl.BlockSpec((BB, D, CJ), lambda b, c: (b, 0, c)),
            pl.BlockSpec((BB, 1, CJ), lambda b, c: (b, 0, c)),
            pl.BlockSpec((BB, 1, D), lambda b, c: (b, 0, 0)),
            pl.BlockSpec((BB, 1, 1), lambda b, c: (b, 0, 0)),
        ],
        out_specs=pl.BlockSpec((BB, D, CJ), lambda b, c: (b, 0, c)),
        out_shape=jax.ShapeDtypeStruct((B, D, J), jnp.float32),
    )(mem_t, cnt, content3, gate3)


def _make_sc_hist(B, W, J):
    info = plsc.get_sparse_core_info()
    nc, ns = info.num_cores, info.num_subcores
    nw = nc * ns
    bpw = B // nw
    mesh = plsc.VectorSubcoreMesh(core_axis_name="c", subcore_axis_name="s")

    @functools.partial(
        pl.kernel,
        out_type=jax.ShapeDtypeStruct((B, 1, J), jnp.float32),
        mesh=mesh,
        compiler_params=pltpu.CompilerParams(needs_layout_passes=False),
        scratch_types=[
            pltpu.VMEM((bpw, W), jnp.int32),
            pltpu.VMEM((J,), jnp.float32),
            pltpu.SemaphoreType.DMA,
            pltpu.SemaphoreType.DMA,
        ],
    )
    def sc_hist(idx_hbm, zeros_hbm, out_hbm, idx_v, counts_v, sem_i, sem_o):
        wid = lax.axis_index("s") * nc + lax.axis_index("c")
        ones16 = jnp.ones((16,), jnp.float32)
        neg16 = jnp.full((16,), -1.0, jnp.float32)
        cp_i = pltpu.async_copy(
            idx_hbm.at[pl.ds(wid * bpw, bpw)], idx_v, sem_i)
        pltpu.sync_copy(zeros_hbm, counts_v)
        cp_i.wait()
        for k in range(bpw):
            b = wid * bpw + k
            for ci in range(W // 16):
                v = idx_v[k, pl.ds(ci * 16, 16)]
                plsc.addupdate_scatter(counts_v, [v], ones16)
            pltpu.async_copy(counts_v, out_hbm.at[b, 0], sem_o).wait()
            if k + 1 < bpw:
                for ci in range(W // 16):
                    v = idx_v[k, pl.ds(ci * 16, 16)]
                    plsc.addupdate_scatter(counts_v, [v], neg16)

    return sc_hist


def kernel(memory_state, write_indices, write_content, gate):
    B, J, D = memory_state.shape
    W = write_indices.shape[1]
    idx = write_indices.astype(jnp.int32)
    zeros = jnp.zeros((J,), jnp.float32)
    counts = _make_sc_hist(B, W, J)(idx, zeros)
    content3 = write_content.reshape(B, 1, D)
    gate3 = gate.reshape(B, 1, 1)
    mem_t = jnp.transpose(memory_state, (0, 2, 1))
    out_t = _tc_apply(mem_t, counts, content3, gate3)
    return jnp.transpose(out_t, (0, 2, 1))
